# baseline (speedup 1.0000x reference)
import functools

import jax
import jax.numpy as jnp
from jax import lax
from jax.experimental import pallas as pl
from jax.experimental.pallas import tpu as pltpu
from jax.experimental.pallas import tpu_sc as plsc

B = 16384
D = 64
L = 16
NCHUNK_D = D // L

_info = plsc.get_sparse_core_info()
NC = _info.num_cores
NS = _info.num_subcores
NW = NC * NS
BPW = B // NW
CH = 128
NCH = BPW // CH


_GATHER_DNUMS = lax.GatherDimensionNumbers(
    offset_dims=(), collapsed_slice_dims=(0,), start_index_map=(0,))


def _lane_perm(x, idx):
    return lax.gather(x, idx[:, None], _GATHER_DNUMS, (1,),
                      mode=lax.GatherScatterMode.PROMISE_IN_BOUNDS)


def _allsum(x):
    lanes = lax.iota(jnp.int32, L)
    for sh in (8, 4, 2, 1):
        x = x + _lane_perm(x, lanes ^ sh)
    return x


def _rsqrt(x):
    i = lax.bitcast_convert_type(x, jnp.int32)
    i = jnp.int32(0x5F3759DF) - lax.shift_right_arithmetic(i, 1)
    y = lax.bitcast_convert_type(i, jnp.float32)
    for _ in range(2):
        y = y * (jnp.float32(1.5) - jnp.float32(0.5) * x * y * y)
    return y


def _body(head_hbm, tail_hbm, rela_hbm, ent_hbm, rel_hbm, out_hbm,
          hidx, tidx, ridx, hrow, trow, rrow, outv, sem):
    wid = lax.axis_index("s") * NC + lax.axis_index("c")
    base = wid * BPW

    pltpu.sync_copy(head_hbm.at[wid], hidx)
    pltpu.sync_copy(tail_hbm.at[wid], tidx)
    pltpu.sync_copy(rela_hbm.at[wid], ridx)

    copies = []
    for c in range(NCH):
        sl = pl.ds(c * CH, CH)
        copies.append(pltpu.async_copy(ent_hbm.at[hidx.at[c]], hrow.at[sl], sem))
        copies.append(pltpu.async_copy(ent_hbm.at[tidx.at[c]], trow.at[sl], sem))
        copies.append(pltpu.async_copy(rel_hbm.at[ridx.at[c]], rrow.at[sl], sem))
    for cp in copies:
        cp.wait()

    lane0 = lax.iota(jnp.int32, L) == 0

    def item(i, carry):
        h = [hrow[i, pl.ds(k * L, L)] for k in range(NCHUNK_D)]
        t = [trow[i, pl.ds(k * L, L)] for k in range(NCHUNK_D)]
        r = [rrow[i, pl.ds(k * L, L)] for k in range(NCHUNK_D)]
        hs = h[0] * h[0]
        ts = t[0] * t[0]
        for k in range(1, NCHUNK_D):
            hs = hs + h[k] * h[k]
            ts = ts + t[k] * t[k]
        ss_h = _allsum(hs)
        ss_t = _allsum(ts)
        rh = _rsqrt(jnp.maximum(ss_h, jnp.float32(1e-24)))
        rt = _rsqrt(jnp.maximum(ss_t, jnp.float32(1e-24)))
        d0 = t[0] * rt - h[0] * rh - r[0]
        acc = d0 * d0
        for k in range(1, NCHUNK_D):
            dk = t[k] * rt - h[k] * rh - r[k]
            acc = acc + dk * dk
        s = _allsum(acc)
        out_i = s * _rsqrt(jnp.maximum(s, jnp.float32(1e-30)))
        plsc.store_scatter(outv, [jnp.full((L,), i, jnp.int32)],
                           out_i, mask=lane0)
        return carry

    lax.fori_loop(0, BPW, item, jnp.int32(0), unroll=4)

    pltpu.sync_copy(outv, out_hbm.at[pl.ds(base, BPW)])


@jax.jit
def _run(head3, tail3, rela3, ent_emb, rela_emb):
    mesh = plsc.VectorSubcoreMesh(core_axis_name="c", subcore_axis_name="s")
    f = functools.partial(
        pl.kernel,
        mesh=mesh,
        compiler_params=pltpu.CompilerParams(
            needs_layout_passes=False, use_tc_tiling_on_sc=False),
        out_type=jax.ShapeDtypeStruct((B,), jnp.float32),
        scratch_types=[
            pltpu.VMEM((NCH, CH), jnp.int32),
            pltpu.VMEM((NCH, CH), jnp.int32),
            pltpu.VMEM((NCH, CH), jnp.int32),
            pltpu.VMEM((BPW, D), jnp.float32),
            pltpu.VMEM((BPW, D), jnp.float32),
            pltpu.VMEM((BPW, D), jnp.float32),
            pltpu.VMEM((BPW,), jnp.float32),
            pltpu.SemaphoreType.DMA,
        ],
    )(_body)
    return f(head3, tail3, rela3, ent_emb, rela_emb)


def kernel(head, tail, rela, ent_emb, rela_emb):
    shape = head.shape
    head3 = head.reshape(NW, NCH, CH).astype(jnp.int32)
    tail3 = tail.reshape(NW, NCH, CH).astype(jnp.int32)
    rela3 = rela.reshape(NW, NCH, CH).astype(jnp.int32)
    out = _run(head3, tail3, rela3, ent_emb, rela_emb)
    return out.reshape(shape)
